# baseline (speedup 1.0000x reference)
import functools

import jax
import jax.numpy as jnp
from jax.experimental import pallas as pl
from jax.experimental.pallas import tpu as pltpu

B, S, D = 4, 8192, 2048
POOL, VOCAB = 48, 500
TOPK = 4
CHUNK = 256


def _tc_kernel(x_ref, wte_ref, w_ref, b_ref, rows_ref, rsim_ref, acc_ref):
    i = pl.program_id(0)
    nsteps = pl.num_programs(0)

    @pl.when(i == 0)
    def _init():
        acc_ref[...] = jnp.zeros_like(acc_ref)

    acc_ref[...] += jnp.sum(x_ref[...], axis=1)

    @pl.when(i == nsteps - 1)
    def _epilogue():
        x_mean = acc_ref[...] * (1.0 / S)
        pk = jax.lax.dot_general(
            w_ref[...], wte_ref[...],
            (((1,), (0,)), ((), ())),
            preferred_element_type=jnp.float32,
        ) + b_ref[...]
        sq = jnp.sum(pk * pk, axis=1, keepdims=True)
        pk_norm = pk * jax.lax.rsqrt(jnp.maximum(sq, 1e-12))
        sim = jax.lax.dot_general(
            x_mean, pk_norm,
            (((1,), (1,)), ((), ())),
            preferred_element_type=jnp.float32,
        )
        rsim_ref[...] = (jnp.sum(sim) * (1.0 / B)).reshape(1, 1)

        R = B * TOPK
        rep = (jax.lax.broadcasted_iota(jnp.int32, (R, B), 0) // TOPK
               == jax.lax.broadcasted_iota(jnp.int32, (R, B), 1)
               ).astype(jnp.float32)
        sim_big = jax.lax.dot_general(
            rep, sim, (((1,), (0,)), ((), ())),
            preferred_element_type=jnp.float32)
        iota = jax.lax.broadcasted_iota(jnp.int32, (R, POOL), 1)
        row_k = jax.lax.broadcasted_iota(jnp.int32, (R, 1), 0) % TOPK
        masked = sim_big
        sel_rows = jnp.zeros((R, 1), jnp.int32)
        for k in range(TOPK):
            m = jnp.max(masked, axis=1, keepdims=True)
            cand = jnp.where(masked == m, iota, POOL + 1)
            amin = jnp.min(cand, axis=1, keepdims=True)
            sel_rows = sel_rows + jnp.where(row_k == k, amin, 0)
            masked = jnp.where(iota == amin, -jnp.inf, masked)
        oh_all = (iota == sel_rows).astype(jnp.float32)
        rows_ref[...] = jax.lax.dot_general(
            oh_all, pk,
            (((1,), (0,)), ((), ())),
            preferred_element_type=jnp.float32,
        )


def kernel(x_embed, wte, W, b):
    nsteps = S // CHUNK
    rows, rsim = pl.pallas_call(
        _tc_kernel,
        grid=(nsteps,),
        in_specs=[
            pl.BlockSpec((B, CHUNK, D), lambda i: (0, i, 0)),
            pl.BlockSpec((VOCAB, D), lambda i: (0, 0)),
            pl.BlockSpec((POOL, VOCAB), lambda i: (0, 0)),
            pl.BlockSpec((POOL, 1), lambda i: (0, 0)),
        ],
        out_specs=[
            pl.BlockSpec((B * TOPK, D), lambda i: (0, 0)),
            pl.BlockSpec((1, 1), lambda i: (0, 0)),
        ],
        out_shape=[
            jax.ShapeDtypeStruct((B * TOPK, D), jnp.float32),
            jax.ShapeDtypeStruct((1, 1), jnp.float32),
        ],
        scratch_shapes=[pltpu.VMEM((B, D), jnp.float32)],
    )(x_embed, wte, W, b.reshape(POOL, 1))
    return rows.reshape(B, TOPK, D), rsim[0, 0]


# speedup vs baseline: 1.0066x; 1.0066x over previous
import functools

import jax
import jax.numpy as jnp
from jax.experimental import pallas as pl
from jax.experimental.pallas import tpu as pltpu

B, S, D = 4, 8192, 2048
POOL, VOCAB = 48, 500
TOPK = 4
CHUNK = 256


def _tc_kernel(x_ref, wte_ref, w_ref, b_ref, rows_ref, rsim_ref,
               acc_ref, pk_ref, pkn_ref):
    i = pl.program_id(0)
    nsteps = pl.num_programs(0)

    @pl.when(i == 0)
    def _init():
        acc_ref[...] = jnp.zeros_like(acc_ref)
        pk = jax.lax.dot_general(
            w_ref[...], wte_ref[...],
            (((1,), (0,)), ((), ())),
            preferred_element_type=jnp.float32,
        ) + b_ref[...]
        sq = jnp.sum(pk * pk, axis=1, keepdims=True)
        pk_ref[...] = pk
        pkn_ref[...] = pk * jax.lax.rsqrt(jnp.maximum(sq, 1e-12))

    acc_ref[...] += jnp.sum(x_ref[...], axis=1)

    @pl.when(i == nsteps - 1)
    def _epilogue():
        x_mean = acc_ref[...] * (1.0 / S)
        pk = pk_ref[...]
        sim = jax.lax.dot_general(
            x_mean, pkn_ref[...],
            (((1,), (1,)), ((), ())),
            preferred_element_type=jnp.float32,
        )
        rsim_ref[...] = (jnp.sum(sim) * (1.0 / B)).reshape(1, 1)

        R = B * TOPK
        rep = (jax.lax.broadcasted_iota(jnp.int32, (R, B), 0) // TOPK
               == jax.lax.broadcasted_iota(jnp.int32, (R, B), 1)
               ).astype(jnp.float32)
        sim_big = jax.lax.dot_general(
            rep, sim, (((1,), (0,)), ((), ())),
            preferred_element_type=jnp.float32)
        iota = jax.lax.broadcasted_iota(jnp.int32, (R, POOL), 1)
        row_k = jax.lax.broadcasted_iota(jnp.int32, (R, 1), 0) % TOPK
        masked = sim_big
        sel_rows = jnp.zeros((R, 1), jnp.int32)
        for k in range(TOPK):
            m = jnp.max(masked, axis=1, keepdims=True)
            cand = jnp.where(masked == m, iota, POOL + 1)
            amin = jnp.min(cand, axis=1, keepdims=True)
            sel_rows = sel_rows + jnp.where(row_k == k, amin, 0)
            masked = jnp.where(iota == amin, -jnp.inf, masked)
        oh_all = (iota == sel_rows).astype(jnp.float32)
        rows_ref[...] = jax.lax.dot_general(
            oh_all, pk,
            (((1,), (0,)), ((), ())),
            preferred_element_type=jnp.float32,
        )


def kernel(x_embed, wte, W, b):
    nsteps = S // CHUNK
    rows, rsim = pl.pallas_call(
        _tc_kernel,
        grid=(nsteps,),
        in_specs=[
            pl.BlockSpec((B, CHUNK, D), lambda i: (0, i, 0)),
            pl.BlockSpec((VOCAB, D), lambda i: (0, 0)),
            pl.BlockSpec((POOL, VOCAB), lambda i: (0, 0)),
            pl.BlockSpec((POOL, 1), lambda i: (0, 0)),
        ],
        out_specs=[
            pl.BlockSpec((B * TOPK, D), lambda i: (0, 0)),
            pl.BlockSpec((1, 1), lambda i: (0, 0)),
        ],
        out_shape=[
            jax.ShapeDtypeStruct((B * TOPK, D), jnp.float32),
            jax.ShapeDtypeStruct((1, 1), jnp.float32),
        ],
        scratch_shapes=[
            pltpu.VMEM((B, D), jnp.float32),
            pltpu.VMEM((POOL, D), jnp.float32),
            pltpu.VMEM((POOL, D), jnp.float32),
        ],
    )(x_embed, wte, W, b.reshape(POOL, 1))
    return rows.reshape(B, TOPK, D), rsim[0, 0]


# speedup vs baseline: 1.0353x; 1.0285x over previous
import functools

import jax
import jax.numpy as jnp
from jax.experimental import pallas as pl
from jax.experimental.pallas import tpu as pltpu

B, S, D = 4, 8192, 2048
POOL, VOCAB = 48, 500
TOPK = 4
CHUNK = 128


def _tc_kernel(x_ref, wte_ref, w_ref, b_ref, rows_ref, rsim_ref,
               acc_ref, pk_ref, pkn_ref):
    i = pl.program_id(0)
    nsteps = pl.num_programs(0)

    @pl.when(i == 0)
    def _init():
        acc_ref[...] = jnp.zeros_like(acc_ref)
        pk = jax.lax.dot_general(
            w_ref[...], wte_ref[...],
            (((1,), (0,)), ((), ())),
            preferred_element_type=jnp.float32,
        ) + b_ref[...]
        sq = jnp.sum(pk * pk, axis=1, keepdims=True)
        pk_ref[...] = pk
        pkn_ref[...] = pk * jax.lax.rsqrt(jnp.maximum(sq, 1e-12))

    acc_ref[...] += jnp.sum(x_ref[...], axis=1)

    @pl.when(i == nsteps - 1)
    def _epilogue():
        x_mean = acc_ref[...] * (1.0 / S)
        pk = pk_ref[...]
        sim = jax.lax.dot_general(
            x_mean, pkn_ref[...],
            (((1,), (1,)), ((), ())),
            preferred_element_type=jnp.float32,
        )
        rsim_ref[...] = (jnp.sum(sim) * (1.0 / B)).reshape(1, 1)

        R = B * TOPK
        rep = (jax.lax.broadcasted_iota(jnp.int32, (R, B), 0) // TOPK
               == jax.lax.broadcasted_iota(jnp.int32, (R, B), 1)
               ).astype(jnp.float32)
        sim_big = jax.lax.dot_general(
            rep, sim, (((1,), (0,)), ((), ())),
            preferred_element_type=jnp.float32)
        iota = jax.lax.broadcasted_iota(jnp.int32, (R, POOL), 1)
        row_k = jax.lax.broadcasted_iota(jnp.int32, (R, 1), 0) % TOPK
        masked = sim_big
        sel_rows = jnp.zeros((R, 1), jnp.int32)
        for k in range(TOPK):
            m = jnp.max(masked, axis=1, keepdims=True)
            cand = jnp.where(masked == m, iota, POOL + 1)
            amin = jnp.min(cand, axis=1, keepdims=True)
            sel_rows = sel_rows + jnp.where(row_k == k, amin, 0)
            masked = jnp.where(iota == amin, -jnp.inf, masked)
        oh_all = (iota == sel_rows).astype(jnp.float32)
        rows_ref[...] = jax.lax.dot_general(
            oh_all, pk,
            (((1,), (0,)), ((), ())),
            preferred_element_type=jnp.float32,
        )


def kernel(x_embed, wte, W, b):
    nsteps = S // CHUNK
    rows, rsim = pl.pallas_call(
        _tc_kernel,
        grid=(nsteps,),
        in_specs=[
            pl.BlockSpec((B, CHUNK, D), lambda i: (0, i, 0)),
            pl.BlockSpec((VOCAB, D), lambda i: (0, 0)),
            pl.BlockSpec((POOL, VOCAB), lambda i: (0, 0)),
            pl.BlockSpec((POOL, 1), lambda i: (0, 0)),
        ],
        out_specs=[
            pl.BlockSpec((B * TOPK, D), lambda i: (0, 0)),
            pl.BlockSpec((1, 1), lambda i: (0, 0)),
        ],
        out_shape=[
            jax.ShapeDtypeStruct((B * TOPK, D), jnp.float32),
            jax.ShapeDtypeStruct((1, 1), jnp.float32),
        ],
        scratch_shapes=[
            pltpu.VMEM((B, D), jnp.float32),
            pltpu.VMEM((POOL, D), jnp.float32),
            pltpu.VMEM((POOL, D), jnp.float32),
        ],
    )(x_embed, wte, W, b.reshape(POOL, 1))
    return rows.reshape(B, TOPK, D), rsim[0, 0]
